# baseline (speedup 1.0000x reference)
import functools

import jax
import jax.numpy as jnp
from jax import lax
from jax.experimental import pallas as pl
from jax.experimental.pallas import tpu as pltpu
from jax.experimental.pallas import tpu_sc as plsc

CHUNK = 128


@functools.lru_cache(maxsize=None)
def _make_gather(B, E):
    info = plsc.get_sparse_core_info()
    nc, ns = info.num_cores, info.num_subcores
    nw = nc * ns
    bpw = B // nw
    nch = bpw // CHUNK
    mesh = plsc.VectorSubcoreMesh(core_axis_name="c", subcore_axis_name="s")

    @functools.partial(
        pl.kernel,
        mesh=mesh,
        out_type=(
            jax.ShapeDtypeStruct((B, E), jnp.float32),
            jax.ShapeDtypeStruct((B, E), jnp.float32),
        ),
        scratch_types=[
            pltpu.VMEM((nch, CHUNK), jnp.int32),
            pltpu.VMEM((nch, CHUNK), jnp.int32),
            pltpu.VMEM((bpw, E), jnp.float32),
            pltpu.VMEM((bpw, E), jnp.float32),
            pltpu.SemaphoreType.DMA,
        ],
        compiler_params=pltpu.CompilerParams(use_tc_tiling_on_sc=False),
    )
    def gather(uids, iids, utab, itab, uout, iout, uidx, iidx, urows, irows, sem):
        wid = lax.axis_index("s") * nc + lax.axis_index("c")
        base = wid * bpw
        for j in range(nch):
            pltpu.sync_copy(uids.at[pl.ds(base + j * CHUNK, CHUNK)], uidx.at[j])
            pltpu.sync_copy(iids.at[pl.ds(base + j * CHUNK, CHUNK)], iidx.at[j])
        copies = []
        for j in range(nch):
            copies.append(
                pltpu.async_copy(
                    utab.at[uidx.at[j]], urows.at[pl.ds(j * CHUNK, CHUNK)], sem
                )
            )
            copies.append(
                pltpu.async_copy(
                    itab.at[iidx.at[j]], irows.at[pl.ds(j * CHUNK, CHUNK)], sem
                )
            )
        for c in copies:
            c.wait()
        pltpu.sync_copy(urows, uout.at[pl.ds(base, bpw)])
        pltpu.sync_copy(irows, iout.at[pl.ds(base, bpw)])

    return gather


def _mlp_body(u_ref, v_ref, wlu_ref, wlv_ref, w1u_ref, w1v_ref, w2_ref,
              wo_ref, b1_ref, b2_ref, scal_ref, out_ref):
    u = u_ref[...]
    v = v_ref[...]
    dot = functools.partial(jnp.dot, preferred_element_type=jnp.float32)
    lin = dot(u, wlu_ref[...]) + dot(v, wlv_ref[...]) + scal_ref[0, 0]
    h = jnp.maximum(dot(u, w1u_ref[...]) + dot(v, w1v_ref[...]) + b1_ref[...], 0.0)
    h = jnp.maximum(dot(h, w2_ref[...]) + b2_ref[...], 0.0)
    n = jax.nn.sigmoid(dot(h, wo_ref[...]) + scal_ref[0, 1])
    a = jax.nn.sigmoid(scal_ref[0, 2])
    out_ref[...] = a * lin + (1.0 - a) * n


def kernel(user_ids, item_ids, user_table, item_table, W_lin, b_lin,
           W1, b1, W2, b2, W_out, b_out, alpha):
    B = user_ids.shape[0]
    E = user_table.shape[1]
    H1 = W1.shape[1]
    H2 = W2.shape[1]

    u_emb, v_emb = _make_gather(B, E)(
        user_ids.astype(jnp.int32), item_ids.astype(jnp.int32),
        user_table, item_table,
    )

    scal = jnp.stack([b_lin[0], b_out[0], alpha[0]]).reshape(1, 3)
    TB = 2048
    grid = (B // TB,)
    full = lambda s: pl.BlockSpec(s, lambda i: (0, 0))
    out = pl.pallas_call(
        _mlp_body,
        grid=grid,
        in_specs=[
            pl.BlockSpec((TB, E), lambda i: (i, 0)),
            pl.BlockSpec((TB, E), lambda i: (i, 0)),
            full((E, 1)),
            full((E, 1)),
            full((E, H1)),
            full((E, H1)),
            full((H1, H2)),
            full((H2, 1)),
            full((1, H1)),
            full((1, H2)),
            full((1, 3)),
        ],
        out_specs=pl.BlockSpec((TB, 1), lambda i: (i, 0)),
        out_shape=jax.ShapeDtypeStruct((B, 1), jnp.float32),
    )(
        u_emb, v_emb,
        W_lin[:E], W_lin[E:],
        W1[:E], W1[E:],
        W2, W_out,
        b1.reshape(1, H1), b2.reshape(1, H2),
        scal,
    )
    return out
